# baseline (speedup 1.0000x reference)
import jax
import jax.numpy as jnp
from jax.experimental import pallas as pl
from jax.experimental.pallas import tpu as pltpu
from jax.experimental.pallas import tpu_sc as plsc

_ROWS = 4


def kernel(x, table):
    b, s = x.shape
    d = table.shape[1]
    mesh = plsc.VectorSubcoreMesh(core_axis_name="c", subcore_axis_name="s")

    @pl.kernel(
        out_type=jax.ShapeDtypeStruct((b, s, d), table.dtype),
        mesh=mesh,
        compiler_params=pltpu.CompilerParams(use_tc_tiling_on_sc=False),
    )
    def gather_kernel(table_hbm, idx_hbm, out_hbm):
        def body(idx_vmem, out_vmem):
            for r in range(_ROWS):
                pltpu.sync_copy(table_hbm.at[idx_vmem.at[r]], out_vmem.at[r])

        pltpu.emit_pipeline(
            body,
            grid=(b // _ROWS,),
            in_specs=[
                pl.BlockSpec((_ROWS, s), lambda i: (i, 0)),
            ],
            out_specs=[
                pl.BlockSpec((_ROWS, s, d), lambda i: (i, 0, 0)),
            ],
            core_axis_name=("c", "s"),
            dimension_semantics=(pltpu.PARALLEL,),
        )(idx_hbm, out_hbm)

    return gather_kernel(table, x)
